# Initial kernel scaffold; baseline (speedup 1.0000x reference)
import jax
import jax.numpy as jnp
from jax.experimental import pallas as pl


def kernel(pred, target):
    raise NotImplementedError("write your pallas kernel here")


# speedup vs baseline: 12.9259x; 12.9259x over previous
import functools

import jax
import jax.numpy as jnp
from jax import lax
from jax.experimental import pallas as pl
from jax.experimental.pallas import tpu as pltpu
from jax.experimental.pallas import tpu_sc as plsc

N = 16 * 1 * 512 * 512
K_KEPT = max(int(N * (1.0 - 0.7)), max(1, 10000))

NC, NS = 2, 16
NW = NC * NS
SHARD = N // NW
CHUNK = 16384
NCHUNK = SHARD // CHUNK

B1 = 2048
B2 = 2048
B3 = 512

_MESH = plsc.VectorSubcoreMesh(core_axis_name="c", subcore_axis_name="s")
_SC_PARAMS = pltpu.CompilerParams(needs_layout_passes=False)


def _wid():
    return lax.axis_index("s") * NC + lax.axis_index("c")


def _lane_iota():
    return lax.broadcasted_iota(jnp.int32, (16,), 0)


def _zero_i32(ref, nwords):
    def body(j, _):
        ref[pl.ds(j * 16, 16)] = jnp.zeros((16,), jnp.int32)
        return 0
    lax.fori_loop(0, nwords // 16, body, 0)


def _reduce_lanes_i32(hist, row, nbins):
    def body(j, _):
        acc = hist[pl.ds(j * 16, 16)]
        for l in range(1, 16):
            acc = acc + hist[pl.ds(l * nbins + j * 16, 16)]
        row[pl.ds(j * 16, 16)] = acc
        return 0
    lax.fori_loop(0, nbins // 16, body, 0)


def _stream_chunks(loss_hbm, bufs, sems, base, process, carry):
    cp = pltpu.async_copy(loss_hbm.at[pl.ds(base, CHUNK)], bufs[0], sems[0])
    for i in range(NCHUNK):
        if i + 1 < NCHUNK:
            nxt = pltpu.async_copy(
                loss_hbm.at[pl.ds(base + (i + 1) * CHUNK, CHUNK)],
                bufs[(i + 1) % 2], sems[(i + 1) % 2])
        cp.wait()
        carry = process(bufs[i % 2], carry)
        if i + 1 < NCHUNK:
            cp = nxt
    return carry


def _loss_body(x_ref, t_ref, o_ref):
    x = x_ref[...]
    t = t_ref[...]
    o_ref[...] = (jnp.maximum(x, 0.0) - x * t
                  + jnp.log1p(jnp.exp(-jnp.abs(x))))


_loss_call = pl.pallas_call(
    _loss_body,
    out_shape=jax.ShapeDtypeStruct((512, 8192), jnp.float32),
    grid=(16,),
    in_specs=[pl.BlockSpec((32, 8192), lambda i: (i, 0)),
              pl.BlockSpec((32, 8192), lambda i: (i, 0))],
    out_specs=pl.BlockSpec((32, 8192), lambda i: (i, 0)),
)


@functools.partial(
    pl.kernel,
    out_type=jax.ShapeDtypeStruct((NW, B1), jnp.int32),
    mesh=_MESH,
    compiler_params=_SC_PARAMS,
    scratch_types=[
        pltpu.VMEM((CHUNK,), jnp.float32),
        pltpu.VMEM((CHUNK,), jnp.float32),
        pltpu.VMEM((16 * B1,), jnp.int32),
        pltpu.VMEM((B1,), jnp.int32),
        pltpu.SemaphoreType.DMA,
        pltpu.SemaphoreType.DMA,
    ],
)
def _hist1_call(loss_hbm, out_cnt, buf0, buf1, hist, row, sem0, sem1):
    wid = _wid()
    base = wid * SHARD
    _zero_i32(hist, 16 * B1)
    lane_base = _lane_iota() * B1
    ones = jnp.ones((16,), jnp.int32)

    def process(buf, carry):
        def body(j, c):
            v = buf[pl.ds(j * 16, 16)]
            bits = plsc.bitcast(v, jnp.int32)
            b = lax.shift_right_logical(bits, 20)
            plsc.addupdate_scatter(hist, [lane_base + b], ones)
            return c
        return lax.fori_loop(0, CHUNK // 16, body, carry)

    _stream_chunks(loss_hbm, (buf0, buf1), (sem0, sem1), base, process, 0)
    _reduce_lanes_i32(hist, row, B1)
    pltpu.sync_copy(row, out_cnt.at[wid])


@functools.partial(
    pl.kernel,
    out_type=(jax.ShapeDtypeStruct((NW, B2), jnp.int32),
              jax.ShapeDtypeStruct((NW, 16), jnp.float32)),
    mesh=_MESH,
    compiler_params=_SC_PARAMS,
    scratch_types=[
        pltpu.VMEM((CHUNK,), jnp.float32),
        pltpu.VMEM((CHUNK,), jnp.float32),
        pltpu.VMEM((16 * B2,), jnp.int32),
        pltpu.VMEM((B2,), jnp.int32),
        pltpu.VMEM((16,), jnp.int32),
        pltpu.VMEM((16,), jnp.float32),
        pltpu.SemaphoreType.DMA,
        pltpu.SemaphoreType.DMA,
    ],
)
def _hist2_call(loss_hbm, d1_hbm, out_cnt, out_sgt,
                buf0, buf1, hist, row, dvec, acc, sem0, sem1):
    wid = _wid()
    base = wid * SHARD
    pltpu.sync_copy(d1_hbm, dvec)
    dv = dvec[...]
    b1 = dv[0]
    _zero_i32(hist, 16 * B2)
    lane_base = _lane_iota() * B2
    ones = jnp.ones((16,), jnp.int32)
    zeros_f = jnp.zeros((16,), jnp.float32)

    def process(buf, carry):
        def body(j, c):
            v = buf[pl.ds(j * 16, 16)]
            bits = plsc.bitcast(v, jnp.int32)
            b = lax.shift_right_logical(bits, 20)
            mid = lax.shift_right_logical(bits, 9) & (B2 - 1)
            plsc.addupdate_scatter(hist, [lane_base + mid], ones, mask=b == b1)
            return c + jnp.where(b > b1, v, zeros_f)
        return lax.fori_loop(0, CHUNK // 16, body, carry)

    accv = _stream_chunks(loss_hbm, (buf0, buf1), (sem0, sem1), base, process,
                          jnp.zeros((16,), jnp.float32))
    acc[...] = accv
    _reduce_lanes_i32(hist, row, B2)
    pltpu.sync_copy(row, out_cnt.at[wid])
    pltpu.sync_copy(acc, out_sgt.at[wid])


@functools.partial(
    pl.kernel,
    out_type=(jax.ShapeDtypeStruct((NW, B3), jnp.int32),
              jax.ShapeDtypeStruct((NW, B3), jnp.float32),
              jax.ShapeDtypeStruct((NW, 16), jnp.float32)),
    mesh=_MESH,
    compiler_params=_SC_PARAMS,
    scratch_types=[
        pltpu.VMEM((CHUNK,), jnp.float32),
        pltpu.VMEM((CHUNK,), jnp.float32),
        pltpu.VMEM((16 * B3,), jnp.int32),
        pltpu.VMEM((16 * B3,), jnp.float32),
        pltpu.VMEM((B3,), jnp.int32),
        pltpu.VMEM((B3,), jnp.float32),
        pltpu.VMEM((16,), jnp.int32),
        pltpu.VMEM((16,), jnp.float32),
        pltpu.SemaphoreType.DMA,
        pltpu.SemaphoreType.DMA,
    ],
)
def _hist3_call(loss_hbm, d2_hbm, out_cnt, out_sum, out_sgt,
                buf0, buf1, chist, shist, crow, srow, dvec, acc, sem0, sem1):
    wid = _wid()
    base = wid * SHARD
    pltpu.sync_copy(d2_hbm, dvec)
    dv = dvec[...]
    b1 = dv[0]
    b2 = dv[1]
    hi = (b1 << 11) | b2
    hi_end = (b1 + 1) << 11
    _zero_i32(chist, 16 * B3)

    def zf(j, _):
        shist[pl.ds(j * 16, 16)] = jnp.zeros((16,), jnp.float32)
        return 0
    lax.fori_loop(0, B3, zf, 0)
    lane_base = _lane_iota() * B3
    ones = jnp.ones((16,), jnp.int32)
    zeros_f = jnp.zeros((16,), jnp.float32)

    def process(buf, carry):
        def body(j, c):
            v = buf[pl.ds(j * 16, 16)]
            bits = plsc.bitcast(v, jnp.int32)
            p = lax.shift_right_logical(bits, 9)
            low = bits & (B3 - 1)
            eq = p == hi
            idx = lane_base + low
            plsc.addupdate_scatter(chist, [idx], ones, mask=eq)
            plsc.addupdate_scatter(shist, [idx], v, mask=eq)
            gt = (p > hi) & (p < hi_end)
            return c + jnp.where(gt, v, zeros_f)
        return lax.fori_loop(0, CHUNK // 16, body, carry)

    accv = _stream_chunks(loss_hbm, (buf0, buf1), (sem0, sem1), base, process,
                          jnp.zeros((16,), jnp.float32))
    acc[...] = accv
    _reduce_lanes_i32(chist, crow, B3)

    def rbody(j, _):
        a = shist[pl.ds(j * 16, 16)]
        for l in range(1, 16):
            a = a + shist[pl.ds(l * B3 + j * 16, 16)]
        srow[pl.ds(j * 16, 16)] = a
        return 0
    lax.fori_loop(0, B3 // 16, rbody, 0)
    pltpu.sync_copy(crow, out_cnt.at[wid])
    pltpu.sync_copy(srow, out_sum.at[wid])
    pltpu.sync_copy(acc, out_sgt.at[wid])


def _suffix_sums_i32(g2d, rows_n, cols_n):
    rows = jnp.sum(g2d, axis=1)
    ii = lax.broadcasted_iota(jnp.int32, (rows_n, rows_n), 0)
    jj = lax.broadcasted_iota(jnp.int32, (rows_n, rows_n), 1)
    r2 = jnp.broadcast_to(rows[None, :], (rows_n, rows_n))
    sr_strict = jnp.sum(jnp.where(jj > ii, r2, 0), axis=1)
    l1 = lax.broadcasted_iota(jnp.int32, (cols_n, cols_n), 0)
    l2 = lax.broadcasted_iota(jnp.int32, (cols_n, cols_n), 1)
    h3 = jnp.broadcast_to(g2d[:, None, :], (rows_n, cols_n, cols_n))
    m3 = jnp.broadcast_to((l2 >= l1)[None, :, :], (rows_n, cols_n, cols_n))
    sw = jnp.sum(jnp.where(m3, h3, 0), axis=2)
    return sw + sr_strict[:, None]


def _flat_iota(rows_n, cols_n):
    return (lax.broadcasted_iota(jnp.int32, (rows_n, cols_n), 0) * cols_n
            + lax.broadcasted_iota(jnp.int32, (rows_n, cols_n), 1))


def _decide1_body(cnt_ref, o_ref):
    g = jnp.sum(cnt_ref[...], axis=0)
    s = _suffix_sums_i32(g, B1 // 16, 16)
    b = jnp.sum((s >= K_KEPT).astype(jnp.int32)) - 1
    fi = _flat_iota(B1 // 16, 16)
    n_above = jnp.sum(jnp.where(fi > b, g, 0))
    r1 = K_KEPT - n_above
    idx = lax.broadcasted_iota(jnp.int32, (16,), 0)
    o_ref[...] = jnp.where(idx == 0, b, jnp.where(idx == 1, r1, 0))


_decide1_call = pl.pallas_call(
    _decide1_body,
    out_shape=jax.ShapeDtypeStruct((16,), jnp.int32),
)


def _decide2_body(cnt_ref, d1_ref, o_ref):
    b1 = d1_ref[0]
    r1 = d1_ref[1]
    g = jnp.sum(cnt_ref[...], axis=0)
    s = _suffix_sums_i32(g, B2 // 16, 16)
    b = jnp.sum((s >= r1).astype(jnp.int32)) - 1
    fi = _flat_iota(B2 // 16, 16)
    n_above = jnp.sum(jnp.where(fi > b, g, 0))
    r2 = r1 - n_above
    idx = lax.broadcasted_iota(jnp.int32, (16,), 0)
    o_ref[...] = jnp.where(
        idx == 0, b1, jnp.where(idx == 1, b, jnp.where(idx == 2, r2, 0)))


_decide2_call = pl.pallas_call(
    _decide2_body,
    out_shape=jax.ShapeDtypeStruct((16,), jnp.int32),
)


def _final_body(cnt_ref, sum_ref, sgt1_ref, sgt2_ref, d2_ref, o_ref):
    b1 = d2_ref[0]
    b2 = d2_ref[1]
    r2 = d2_ref[2]
    g = jnp.sum(cnt_ref[...], axis=0)
    sm = jnp.sum(sum_ref[...], axis=0)
    s = _suffix_sums_i32(g, B3 // 16, 16)
    b3 = jnp.sum((s >= r2).astype(jnp.int32)) - 1
    fi = _flat_iota(B3 // 16, 16)
    n_above3 = jnp.sum(jnp.where(fi > b3, g, 0))
    s_above3 = jnp.sum(jnp.where(fi > b3, sm, 0.0))
    m = r2 - n_above3
    tau_bits = (b1 << 20) | (b2 << 9) | b3
    tau = lax.bitcast_convert_type(tau_bits, jnp.float32)
    total = (jnp.sum(sgt1_ref[...]) + jnp.sum(sgt2_ref[...]) + s_above3
             + m.astype(jnp.float32) * tau)
    o_ref[...] = (total * jnp.float32(1.0 / K_KEPT))[None]


_final_call = pl.pallas_call(
    _final_body,
    out_shape=jax.ShapeDtypeStruct((1,), jnp.float32),
)


def kernel(pred, target):
    x = pred.reshape(512, 8192)
    t = target.reshape(512, 8192)
    loss = _loss_call(x, t).reshape(N)
    cnt1 = _hist1_call(loss)
    d1 = _decide1_call(cnt1.reshape(NW, B1 // 16, 16))
    cnt2, sgt1 = _hist2_call(loss, d1)
    d2 = _decide2_call(cnt2.reshape(NW, B2 // 16, 16), d1)
    cnt3, sum3, sgt2 = _hist3_call(loss, d2)
    out = _final_call(cnt3.reshape(NW, B3 // 16, 16),
                      sum3.reshape(NW, B3 // 16, 16), sgt1, sgt2, d2)
    return out.reshape(())


# speedup vs baseline: 24.2327x; 1.8747x over previous
import functools

import jax
import jax.numpy as jnp
from jax import lax
from jax.experimental import pallas as pl
from jax.experimental.pallas import tpu as pltpu
from jax.experimental.pallas import tpu_sc as plsc

N = 16 * 1 * 512 * 512
K_KEPT = max(int(N * (1.0 - 0.7)), max(1, 10000))

NC, NS = 2, 16
NW = NC * NS
ROWS, COLS = 8192, 512
SHARD_ROWS = ROWS // NW
CHUNK_ROWS = 32
NCHUNK = SHARD_ROWS // CHUNK_ROWS
U = 8

B1 = 32768
B2 = 65536

_MESH = plsc.VectorSubcoreMesh(core_axis_name="c", subcore_axis_name="s")
_SC_PARAMS = pltpu.CompilerParams(needs_layout_passes=False)


def _wid():
    return lax.axis_index("s") * NC + lax.axis_index("c")


def _zero_i32(ref, nwords):
    def body(j, _):
        for u in range(U):
            ref[pl.ds(j * (16 * U) + u * 16, 16)] = jnp.zeros((16,), jnp.int32)
        return 0
    lax.fori_loop(0, nwords // (16 * U), body, 0)


def _stream_chunks(loss_hbm, bufs, sems, row0, process, carry):
    cp = pltpu.async_copy(loss_hbm.at[pl.ds(row0, CHUNK_ROWS)], bufs[0], sems[0])
    for i in range(NCHUNK):
        if i + 1 < NCHUNK:
            nxt = pltpu.async_copy(
                loss_hbm.at[pl.ds(row0 + (i + 1) * CHUNK_ROWS, CHUNK_ROWS)],
                bufs[(i + 1) % 2], sems[(i + 1) % 2])
        cp.wait()
        carry = process(bufs[i % 2], carry)
        if i + 1 < NCHUNK:
            cp = nxt
    return carry


def _loss_body(x_ref, t_ref, o_ref):
    x = x_ref[...]
    t = t_ref[...]
    o_ref[...] = (jnp.maximum(x, 0.0) - x * t
                  + jnp.log1p(jnp.exp(-jnp.abs(x))))


_loss_call = pl.pallas_call(
    _loss_body,
    out_shape=jax.ShapeDtypeStruct((ROWS, COLS), jnp.float32),
    grid=(16,),
    in_specs=[pl.BlockSpec((ROWS // 16, COLS), lambda i: (i, 0)),
              pl.BlockSpec((ROWS // 16, COLS), lambda i: (i, 0))],
    out_specs=pl.BlockSpec((ROWS // 16, COLS), lambda i: (i, 0)),
)


@functools.partial(
    pl.kernel,
    out_type=jax.ShapeDtypeStruct((NW, B1), jnp.int32),
    mesh=_MESH,
    compiler_params=_SC_PARAMS,
    scratch_types=[
        pltpu.VMEM((CHUNK_ROWS, COLS), jnp.float32),
        pltpu.VMEM((CHUNK_ROWS, COLS), jnp.float32),
        pltpu.VMEM((B1,), jnp.int32),
        pltpu.SemaphoreType.DMA,
        pltpu.SemaphoreType.DMA,
    ],
)
def _hist1_call(loss_hbm, out_cnt, buf0, buf1, hist, sem0, sem1):
    wid = _wid()
    row0 = wid * SHARD_ROWS
    _zero_i32(hist, B1)
    ones = jnp.ones((16,), jnp.int32)

    def process(buf, carry):
        def rbody(r, c):
            for u in range(COLS // 16):
                v = buf[r, pl.ds(u * 16, 16)]
                bits = plsc.bitcast(v, jnp.int32)
                plsc.addupdate_scatter(
                    hist, [lax.shift_right_logical(bits, 16)], ones)
            return c
        return lax.fori_loop(0, CHUNK_ROWS, rbody, carry)

    _stream_chunks(loss_hbm, (buf0, buf1), (sem0, sem1), row0, process, 0)
    pltpu.sync_copy(hist, out_cnt.at[wid])


@functools.partial(
    pl.kernel,
    out_type=(jax.ShapeDtypeStruct((NW, B2), jnp.int32),
              jax.ShapeDtypeStruct((NW, 16), jnp.float32)),
    mesh=_MESH,
    compiler_params=_SC_PARAMS,
    scratch_types=[
        pltpu.VMEM((CHUNK_ROWS, COLS), jnp.float32),
        pltpu.VMEM((CHUNK_ROWS, COLS), jnp.float32),
        pltpu.VMEM((B2,), jnp.int32),
        pltpu.VMEM((16,), jnp.int32),
        pltpu.VMEM((16,), jnp.float32),
        pltpu.SemaphoreType.DMA,
        pltpu.SemaphoreType.DMA,
    ],
)
def _hist2_call(loss_hbm, d1_hbm, out_cnt, out_sgt,
                buf0, buf1, hist, dvec, acc, sem0, sem1):
    wid = _wid()
    row0 = wid * SHARD_ROWS
    pltpu.sync_copy(d1_hbm, dvec)
    dv = dvec[...]
    b1 = dv[0]
    _zero_i32(hist, B2)
    ones = jnp.ones((16,), jnp.int32)
    zeros_f = jnp.zeros((16,), jnp.float32)

    def process(buf, carry):
        def rbody(r, c):
            for u in range(COLS // 16):
                v = buf[r, pl.ds(u * 16, 16)]
                bits = plsc.bitcast(v, jnp.int32)
                hi = lax.shift_right_logical(bits, 16)
                low = bits & (B2 - 1)
                plsc.addupdate_scatter(hist, [low], ones, mask=hi == b1)
                c = c + jnp.where(hi > b1, v, zeros_f)
            return c
        return lax.fori_loop(0, CHUNK_ROWS, rbody, carry)

    accv = _stream_chunks(loss_hbm, (buf0, buf1), (sem0, sem1), row0, process,
                          jnp.zeros((16,), jnp.float32))
    acc[...] = accv
    pltpu.sync_copy(hist, out_cnt.at[wid])
    pltpu.sync_copy(acc, out_sgt.at[wid])


def _suffix_incl(g, rows_n):
    c1 = lax.broadcasted_iota(jnp.int32, (128, 128), 0)
    c2 = lax.broadcasted_iota(jnp.int32, (128, 128), 1)
    m = (c1 >= c2).astype(jnp.float32)
    sw = jax.lax.dot(g, m)
    rows = jnp.sum(g, axis=1)
    i1 = lax.broadcasted_iota(jnp.int32, (rows_n, rows_n), 0)
    i2 = lax.broadcasted_iota(jnp.int32, (rows_n, rows_n), 1)
    r2 = jnp.broadcast_to(rows[None, :], (rows_n, rows_n))
    sr = jnp.sum(jnp.where(i2 > i1, r2, 0.0), axis=1)
    return sw + sr[:, None]


def _flat_iota(rows_n):
    return (lax.broadcasted_iota(jnp.int32, (rows_n, 128), 0) * 128
            + lax.broadcasted_iota(jnp.int32, (rows_n, 128), 1))


def _decide1_body(cnt_ref, o_ref):
    g = jnp.sum(cnt_ref[...], axis=0)
    s = _suffix_incl(g.astype(jnp.float32), B1 // 128)
    b1 = jnp.sum((s >= jnp.float32(K_KEPT)).astype(jnp.int32)) - 1
    fi = _flat_iota(B1 // 128)
    n_above = jnp.sum(jnp.where(fi > b1, g, 0))
    r1 = K_KEPT - n_above
    idx = lax.broadcasted_iota(jnp.int32, (16,), 0)
    o_ref[...] = jnp.where(idx == 0, b1, jnp.where(idx == 1, r1, 0))


_decide1_call = pl.pallas_call(
    _decide1_body,
    out_shape=jax.ShapeDtypeStruct((16,), jnp.int32),
)


def _final_body(cnt_ref, sgt_ref, d1_ref, o_ref):
    b1 = d1_ref[0]
    r1 = d1_ref[1]
    g = jnp.sum(cnt_ref[...], axis=0)
    gf = g.astype(jnp.float32)
    s = _suffix_incl(gf, B2 // 128)
    b2 = jnp.sum((s >= r1.astype(jnp.float32)).astype(jnp.int32)) - 1
    fi = _flat_iota(B2 // 128)
    n_above = jnp.sum(jnp.where(fi > b2, g, 0))
    m = r1 - n_above
    vj = lax.bitcast_convert_type((b1 << 16) | fi, jnp.float32)
    sum_above = jnp.sum(jnp.where(fi > b2, gf, 0.0) * vj)
    tau = jnp.sum(jnp.where(fi == b2, vj, 0.0))
    total = jnp.sum(sgt_ref[...]) + sum_above + m.astype(jnp.float32) * tau
    o_ref[...] = (total * jnp.float32(1.0 / K_KEPT))[None]


_final_call = pl.pallas_call(
    _final_body,
    out_shape=jax.ShapeDtypeStruct((1,), jnp.float32),
)


def kernel(pred, target):
    x = pred.reshape(ROWS, COLS)
    t = target.reshape(ROWS, COLS)
    loss = _loss_call(x, t)
    cnt1 = _hist1_call(loss)
    d1 = _decide1_call(cnt1.reshape(NW, B1 // 128, 128))
    cnt2, sgt = _hist2_call(loss, d1)
    out = _final_call(cnt2.reshape(NW, B2 // 128, 128), sgt, d1)
    return out.reshape(())


# speedup vs baseline: 26.3184x; 1.0861x over previous
import functools

import jax
import jax.numpy as jnp
from jax import lax
from jax.experimental import pallas as pl
from jax.experimental.pallas import tpu as pltpu
from jax.experimental.pallas import tpu_sc as plsc

N = 16 * 1 * 512 * 512
K_KEPT = max(int(N * (1.0 - 0.7)), max(1, 10000))

NC, NS = 2, 16
NW = NC * NS
ROWS, COLS = 8192, 512
SHARD_ROWS = ROWS // NW
CHUNK_ROWS = 32
NCHUNK = SHARD_ROWS // CHUNK_ROWS
U = 8

B1 = 32768
B2 = 65536

_MESH = plsc.VectorSubcoreMesh(core_axis_name="c", subcore_axis_name="s")
_SC_PARAMS = pltpu.CompilerParams(needs_layout_passes=False)


def _wid():
    return lax.axis_index("s") * NC + lax.axis_index("c")


def _zero2d_i32(ref, nrows):
    def body(r, _):
        for u in range(8):
            ref[r, pl.ds(u * 16, 16)] = jnp.zeros((16,), jnp.int32)
        return 0
    lax.fori_loop(0, nrows, body, 0)


def _stream_chunks(loss_hbm, bufs, sems, row0, process, carry):
    cp = pltpu.async_copy(loss_hbm.at[pl.ds(row0, CHUNK_ROWS)], bufs[0], sems[0])
    for i in range(NCHUNK):
        if i + 1 < NCHUNK:
            nxt = pltpu.async_copy(
                loss_hbm.at[pl.ds(row0 + (i + 1) * CHUNK_ROWS, CHUNK_ROWS)],
                bufs[(i + 1) % 2], sems[(i + 1) % 2])
        cp.wait()
        carry = process(bufs[i % 2], carry)
        if i + 1 < NCHUNK:
            cp = nxt
    return carry


def _loss_body(x_ref, t_ref, o_ref):
    x = x_ref[...]
    t = t_ref[...]
    o_ref[...] = (jnp.maximum(x, 0.0) - x * t
                  + jnp.log1p(jnp.exp(-jnp.abs(x))))


_loss_call = pl.pallas_call(
    _loss_body,
    out_shape=jax.ShapeDtypeStruct((ROWS, COLS), jnp.float32),
    grid=(16,),
    in_specs=[pl.BlockSpec((ROWS // 16, COLS), lambda i: (i, 0)),
              pl.BlockSpec((ROWS // 16, COLS), lambda i: (i, 0))],
    out_specs=pl.BlockSpec((ROWS // 16, COLS), lambda i: (i, 0)),
)


@functools.partial(
    pl.kernel,
    out_type=jax.ShapeDtypeStruct((NW, B1 // 128, 128), jnp.int32),
    mesh=_MESH,
    compiler_params=_SC_PARAMS,
    scratch_types=[
        pltpu.VMEM((CHUNK_ROWS, COLS), jnp.float32),
        pltpu.VMEM((CHUNK_ROWS, COLS), jnp.float32),
        pltpu.VMEM((B1 // 128, 128), jnp.int32),
        pltpu.SemaphoreType.DMA,
        pltpu.SemaphoreType.DMA,
    ],
)
def _hist1_call(loss_hbm, out_cnt, buf0, buf1, hist, sem0, sem1):
    wid = _wid()
    row0 = wid * SHARD_ROWS
    _zero2d_i32(hist, B1 // 128)
    ones = jnp.ones((16,), jnp.int32)

    def process(buf, carry):
        def rbody(r, c):
            for u in range(COLS // 16):
                v = buf[r, pl.ds(u * 16, 16)]
                bits = plsc.bitcast(v, jnp.int32)
                b = lax.shift_right_logical(bits, 16)
                plsc.addupdate_scatter(
                    hist, [lax.shift_right_logical(b, 7), b & 127], ones)
            return c
        return lax.fori_loop(0, CHUNK_ROWS, rbody, carry)

    _stream_chunks(loss_hbm, (buf0, buf1), (sem0, sem1), row0, process, 0)
    pltpu.sync_copy(hist, out_cnt.at[wid])


@functools.partial(
    pl.kernel,
    out_type=(jax.ShapeDtypeStruct((NW, B2 // 128, 128), jnp.int32),
              jax.ShapeDtypeStruct((NW, 16), jnp.float32)),
    mesh=_MESH,
    compiler_params=_SC_PARAMS,
    scratch_types=[
        pltpu.VMEM((CHUNK_ROWS, COLS), jnp.float32),
        pltpu.VMEM((CHUNK_ROWS, COLS), jnp.float32),
        pltpu.VMEM((B2 // 128, 128), jnp.int32),
        pltpu.VMEM((16,), jnp.int32),
        pltpu.VMEM((16,), jnp.float32),
        pltpu.SemaphoreType.DMA,
        pltpu.SemaphoreType.DMA,
    ],
)
def _hist2_call(loss_hbm, d1_hbm, out_cnt, out_sgt,
                buf0, buf1, hist, dvec, acc, sem0, sem1):
    wid = _wid()
    row0 = wid * SHARD_ROWS
    pltpu.sync_copy(d1_hbm, dvec)
    dv = dvec[...]
    b1 = dv[0]
    _zero2d_i32(hist, B2 // 128)
    ones = jnp.ones((16,), jnp.int32)
    zeros_f = jnp.zeros((16,), jnp.float32)

    def process(buf, carry):
        def rbody(r, cs):
            cs = list(cs)
            for u in range(COLS // 16):
                v = buf[r, pl.ds(u * 16, 16)]
                bits = plsc.bitcast(v, jnp.int32)
                hi = lax.shift_right_logical(bits, 16)
                low = bits & (B2 - 1)
                plsc.addupdate_scatter(
                    hist, [lax.shift_right_logical(low, 7), low & 127],
                    ones, mask=hi == b1)
                cs[u % 4] = cs[u % 4] + jnp.where(hi > b1, v, zeros_f)
            return tuple(cs)
        return lax.fori_loop(0, CHUNK_ROWS, rbody, carry)

    z4 = (jnp.zeros((16,), jnp.float32),) * 4
    accs = _stream_chunks(loss_hbm, (buf0, buf1), (sem0, sem1), row0, process,
                          z4)
    acc[...] = accs[0] + accs[1] + accs[2] + accs[3]
    pltpu.sync_copy(hist, out_cnt.at[wid])
    pltpu.sync_copy(acc, out_sgt.at[wid])


def _suffix_incl(g, rows_n):
    c1 = lax.broadcasted_iota(jnp.int32, (128, 128), 0)
    c2 = lax.broadcasted_iota(jnp.int32, (128, 128), 1)
    m = (c1 >= c2).astype(jnp.float32)
    sw = jax.lax.dot(g, m)
    rows = jnp.sum(g, axis=1)
    i1 = lax.broadcasted_iota(jnp.int32, (rows_n, rows_n), 0)
    i2 = lax.broadcasted_iota(jnp.int32, (rows_n, rows_n), 1)
    r2 = jnp.broadcast_to(rows[None, :], (rows_n, rows_n))
    sr = jnp.sum(jnp.where(i2 > i1, r2, 0.0), axis=1)
    return sw + sr[:, None]


def _flat_iota(rows_n):
    return (lax.broadcasted_iota(jnp.int32, (rows_n, 128), 0) * 128
            + lax.broadcasted_iota(jnp.int32, (rows_n, 128), 1))


def _decide1_body(cnt_ref, o_ref):
    g = jnp.sum(cnt_ref[...], axis=0)
    s = _suffix_incl(g.astype(jnp.float32), B1 // 128)
    b1 = jnp.sum((s >= jnp.float32(K_KEPT)).astype(jnp.int32)) - 1
    fi = _flat_iota(B1 // 128)
    n_above = jnp.sum(jnp.where(fi > b1, g, 0))
    r1 = K_KEPT - n_above
    idx = lax.broadcasted_iota(jnp.int32, (16,), 0)
    o_ref[...] = jnp.where(idx == 0, b1, jnp.where(idx == 1, r1, 0))


_decide1_call = pl.pallas_call(
    _decide1_body,
    out_shape=jax.ShapeDtypeStruct((16,), jnp.int32),
)


def _final_body(cnt_ref, sgt_ref, d1_ref, o_ref):
    b1 = d1_ref[0]
    r1 = d1_ref[1]
    g = jnp.sum(cnt_ref[...], axis=0)
    gf = g.astype(jnp.float32)
    s = _suffix_incl(gf, B2 // 128)
    b2 = jnp.sum((s >= r1.astype(jnp.float32)).astype(jnp.int32)) - 1
    fi = _flat_iota(B2 // 128)
    n_above = jnp.sum(jnp.where(fi > b2, g, 0))
    m = r1 - n_above
    vj = lax.bitcast_convert_type((b1 << 16) | fi, jnp.float32)
    sum_above = jnp.sum(jnp.where(fi > b2, gf, 0.0) * vj)
    tau = jnp.sum(jnp.where(fi == b2, vj, 0.0))
    total = jnp.sum(sgt_ref[...]) + sum_above + m.astype(jnp.float32) * tau
    o_ref[...] = (total * jnp.float32(1.0 / K_KEPT))[None]


_final_call = pl.pallas_call(
    _final_body,
    out_shape=jax.ShapeDtypeStruct((1,), jnp.float32),
)


def kernel(pred, target):
    x = pred.reshape(ROWS, COLS)
    t = target.reshape(ROWS, COLS)
    loss = _loss_call(x, t)
    cnt1 = _hist1_call(loss)
    d1 = _decide1_call(cnt1)
    cnt2, sgt = _hist2_call(loss, d1)
    out = _final_call(cnt2, sgt, d1)
    return out.reshape(())


# speedup vs baseline: 52.2692x; 1.9860x over previous
import functools

import jax
import jax.numpy as jnp
from jax import lax
from jax.experimental import pallas as pl
from jax.experimental.pallas import tpu as pltpu
from jax.experimental.pallas import tpu_sc as plsc

N = 16 * 1 * 512 * 512
K_KEPT = max(int(N * (1.0 - 0.7)), max(1, 10000))

NC, NS = 2, 16
NW = NC * NS
ROWS, COLS = 8192, 512
SHARD_ROWS = ROWS // NW
CHUNK_ROWS = 32
NCHUNK = SHARD_ROWS // CHUNK_ROWS
U = 8

B1 = 32768
B2 = 65536

_MESH = plsc.VectorSubcoreMesh(core_axis_name="c", subcore_axis_name="s")
_SC_PARAMS = pltpu.CompilerParams(needs_layout_passes=False)


def _wid():
    return lax.axis_index("s") * NC + lax.axis_index("c")


def _zero2d_i32(ref, nrows):
    def body(r, _):
        for u in range(8):
            ref[r, pl.ds(u * 16, 16)] = jnp.zeros((16,), jnp.int32)
        return 0
    lax.fori_loop(0, nrows, body, 0)


def _stream_chunks(loss_hbm, bufs, sems, row0, process, carry):
    cp = pltpu.async_copy(loss_hbm.at[pl.ds(row0, CHUNK_ROWS)], bufs[0], sems[0])
    for i in range(NCHUNK):
        if i + 1 < NCHUNK:
            nxt = pltpu.async_copy(
                loss_hbm.at[pl.ds(row0 + (i + 1) * CHUNK_ROWS, CHUNK_ROWS)],
                bufs[(i + 1) % 2], sems[(i + 1) % 2])
        cp.wait()
        carry = process(bufs[i % 2], carry)
        if i + 1 < NCHUNK:
            cp = nxt
    return carry


def _loss_body(x_ref, t_ref, o_ref):
    x = x_ref[...]
    t = t_ref[...]
    o_ref[...] = (jnp.maximum(x, 0.0) - x * t
                  + jnp.log1p(jnp.exp(-jnp.abs(x))))


_loss_call = pl.pallas_call(
    _loss_body,
    out_shape=jax.ShapeDtypeStruct((ROWS, COLS), jnp.float32),
    grid=(16,),
    in_specs=[pl.BlockSpec((ROWS // 16, COLS), lambda i: (i, 0)),
              pl.BlockSpec((ROWS // 16, COLS), lambda i: (i, 0))],
    out_specs=pl.BlockSpec((ROWS // 16, COLS), lambda i: (i, 0)),
)


@functools.partial(
    pl.kernel,
    out_type=jax.ShapeDtypeStruct((NW, B1 // 128, 128), jnp.int32),
    mesh=_MESH,
    compiler_params=_SC_PARAMS,
    scratch_types=[
        pltpu.VMEM((CHUNK_ROWS, COLS), jnp.float32),
        pltpu.VMEM((CHUNK_ROWS, COLS), jnp.float32),
        pltpu.VMEM((B1 // 128, 128), jnp.int32),
        pltpu.SemaphoreType.DMA,
        pltpu.SemaphoreType.DMA,
    ],
)
def _hist1_call(loss_hbm, out_cnt, buf0, buf1, hist, sem0, sem1):
    wid = _wid()
    row0 = wid * SHARD_ROWS
    _zero2d_i32(hist, B1 // 128)
    ones = jnp.ones((16,), jnp.int32)

    def process(buf, carry):
        def qbody(q):
            r = lax.shift_right_logical(q, 2)
            c0 = (q & 3) * 128
            for u in range(8):
                v = buf[r, pl.ds(c0 + u * 16, 16)]
                bits = plsc.bitcast(v, jnp.int32)
                b = lax.shift_right_logical(bits, 16)
                plsc.addupdate_scatter(
                    hist, [lax.shift_right_logical(b, 7), b & 127], ones)
        plsc.parallel_loop(0, CHUNK_ROWS * 4, 1, unroll=2)(qbody)
        return carry

    _stream_chunks(loss_hbm, (buf0, buf1), (sem0, sem1), row0, process, 0)
    pltpu.sync_copy(hist, out_cnt.at[wid])


@functools.partial(
    pl.kernel,
    out_type=(jax.ShapeDtypeStruct((NW, B2 // 128, 128), jnp.int32),
              jax.ShapeDtypeStruct((NW, 16), jnp.float32)),
    mesh=_MESH,
    compiler_params=_SC_PARAMS,
    scratch_types=[
        pltpu.VMEM((CHUNK_ROWS, COLS), jnp.float32),
        pltpu.VMEM((CHUNK_ROWS, COLS), jnp.float32),
        pltpu.VMEM((B2 // 128, 128), jnp.int32),
        pltpu.VMEM((16,), jnp.int32),
        pltpu.VMEM((16,), jnp.float32),
        pltpu.SemaphoreType.DMA,
        pltpu.SemaphoreType.DMA,
    ],
)
def _hist2_call(loss_hbm, d1_hbm, out_cnt, out_sgt,
                buf0, buf1, hist, dvec, acc, sem0, sem1):
    wid = _wid()
    row0 = wid * SHARD_ROWS
    pltpu.sync_copy(d1_hbm, dvec)
    dv = dvec[...]
    b1 = dv[0]
    _zero2d_i32(hist, B2 // 128)
    ones = jnp.ones((16,), jnp.int32)
    zeros_f = jnp.zeros((16,), jnp.float32)

    def process(buf, carry):
        def qbody(q, cs):
            cs = list(cs)
            r = lax.shift_right_logical(q, 2)
            c0 = (q & 3) * 128
            for u in range(8):
                v = buf[r, pl.ds(c0 + u * 16, 16)]
                bits = plsc.bitcast(v, jnp.int32)
                hi = lax.shift_right_logical(bits, 16)
                low = bits & (B2 - 1)
                plsc.addupdate_scatter(
                    hist, [lax.shift_right_logical(low, 7), low & 127],
                    ones, mask=hi == b1)
                cs[u % 4] = cs[u % 4] + jnp.where(hi > b1, v, zeros_f)
            return tuple(cs)
        return plsc.parallel_loop(0, CHUNK_ROWS * 4, 1, unroll=2,
                                  carry=carry)(qbody)

    z4 = (jnp.zeros((16,), jnp.float32),) * 4
    accs = _stream_chunks(loss_hbm, (buf0, buf1), (sem0, sem1), row0, process,
                          z4)
    acc[...] = accs[0] + accs[1] + accs[2] + accs[3]
    pltpu.sync_copy(hist, out_cnt.at[wid])
    pltpu.sync_copy(acc, out_sgt.at[wid])


def _suffix_incl(g, rows_n):
    c1 = lax.broadcasted_iota(jnp.int32, (128, 128), 0)
    c2 = lax.broadcasted_iota(jnp.int32, (128, 128), 1)
    m = (c1 >= c2).astype(jnp.float32)
    sw = jax.lax.dot(g, m)
    rows = jnp.sum(g, axis=1)
    i1 = lax.broadcasted_iota(jnp.int32, (rows_n, rows_n), 0)
    i2 = lax.broadcasted_iota(jnp.int32, (rows_n, rows_n), 1)
    r2 = jnp.broadcast_to(rows[None, :], (rows_n, rows_n))
    sr = jnp.sum(jnp.where(i2 > i1, r2, 0.0), axis=1)
    return sw + sr[:, None]


def _flat_iota(rows_n):
    return (lax.broadcasted_iota(jnp.int32, (rows_n, 128), 0) * 128
            + lax.broadcasted_iota(jnp.int32, (rows_n, 128), 1))


def _decide1_body(cnt_ref, o_ref):
    g = jnp.sum(cnt_ref[...], axis=0)
    s = _suffix_incl(g.astype(jnp.float32), B1 // 128)
    b1 = jnp.sum((s >= jnp.float32(K_KEPT)).astype(jnp.int32)) - 1
    fi = _flat_iota(B1 // 128)
    n_above = jnp.sum(jnp.where(fi > b1, g, 0))
    r1 = K_KEPT - n_above
    idx = lax.broadcasted_iota(jnp.int32, (16,), 0)
    o_ref[...] = jnp.where(idx == 0, b1, jnp.where(idx == 1, r1, 0))


_decide1_call = pl.pallas_call(
    _decide1_body,
    out_shape=jax.ShapeDtypeStruct((16,), jnp.int32),
)


def _final_body(cnt_ref, sgt_ref, d1_ref, o_ref):
    b1 = d1_ref[0]
    r1 = d1_ref[1]
    g = jnp.sum(cnt_ref[...], axis=0)
    gf = g.astype(jnp.float32)
    s = _suffix_incl(gf, B2 // 128)
    b2 = jnp.sum((s >= r1.astype(jnp.float32)).astype(jnp.int32)) - 1
    fi = _flat_iota(B2 // 128)
    n_above = jnp.sum(jnp.where(fi > b2, g, 0))
    m = r1 - n_above
    vj = lax.bitcast_convert_type((b1 << 16) | fi, jnp.float32)
    sum_above = jnp.sum(jnp.where(fi > b2, gf, 0.0) * vj)
    tau = jnp.sum(jnp.where(fi == b2, vj, 0.0))
    total = jnp.sum(sgt_ref[...]) + sum_above + m.astype(jnp.float32) * tau
    o_ref[...] = (total * jnp.float32(1.0 / K_KEPT))[None]


_final_call = pl.pallas_call(
    _final_body,
    out_shape=jax.ShapeDtypeStruct((1,), jnp.float32),
)


def kernel(pred, target):
    x = pred.reshape(ROWS, COLS)
    t = target.reshape(ROWS, COLS)
    loss = _loss_call(x, t)
    cnt1 = _hist1_call(loss)
    d1 = _decide1_call(cnt1)
    cnt2, sgt = _hist2_call(loss, d1)
    out = _final_call(cnt2, sgt, d1)
    return out.reshape(())


# speedup vs baseline: 52.4467x; 1.0034x over previous
import functools

import jax
import jax.numpy as jnp
from jax import lax
from jax.experimental import pallas as pl
from jax.experimental.pallas import tpu as pltpu
from jax.experimental.pallas import tpu_sc as plsc

N = 16 * 1 * 512 * 512
K_KEPT = max(int(N * (1.0 - 0.7)), max(1, 10000))

NC, NS = 2, 16
NW = NC * NS
ROWS, COLS = 8192, 512
SHARD_ROWS = ROWS // NW
CHUNK_ROWS = 32
NCHUNK = SHARD_ROWS // CHUNK_ROWS
U = 8

B1 = 32768
B2 = 65536

_MESH = plsc.VectorSubcoreMesh(core_axis_name="c", subcore_axis_name="s")
_SC_PARAMS = pltpu.CompilerParams(needs_layout_passes=False)


def _wid():
    return lax.axis_index("s") * NC + lax.axis_index("c")


def _zero2d_i32(ref, nrows):
    def body(r, _):
        for u in range(8):
            ref[r, pl.ds(u * 16, 16)] = jnp.zeros((16,), jnp.int32)
        return 0
    lax.fori_loop(0, nrows, body, 0)


def _stream_chunks(loss_hbm, bufs, sems, row0, process, carry):
    cp = pltpu.async_copy(loss_hbm.at[pl.ds(row0, CHUNK_ROWS)], bufs[0], sems[0])
    for i in range(NCHUNK):
        if i + 1 < NCHUNK:
            nxt = pltpu.async_copy(
                loss_hbm.at[pl.ds(row0 + (i + 1) * CHUNK_ROWS, CHUNK_ROWS)],
                bufs[(i + 1) % 2], sems[(i + 1) % 2])
        cp.wait()
        carry = process(bufs[i % 2], carry)
        if i + 1 < NCHUNK:
            cp = nxt
    return carry


def _loss_body(x_ref, t_ref, o_ref):
    x = x_ref[...]
    t = t_ref[...]
    o_ref[...] = (jnp.maximum(x, 0.0) - x * t
                  + jnp.log1p(jnp.exp(-jnp.abs(x))))


_loss_call = pl.pallas_call(
    _loss_body,
    out_shape=jax.ShapeDtypeStruct((ROWS, COLS), jnp.float32),
    grid=(16,),
    in_specs=[pl.BlockSpec((ROWS // 16, COLS), lambda i: (i, 0)),
              pl.BlockSpec((ROWS // 16, COLS), lambda i: (i, 0))],
    out_specs=pl.BlockSpec((ROWS // 16, COLS), lambda i: (i, 0)),
)


@functools.partial(
    pl.kernel,
    out_type=jax.ShapeDtypeStruct((NW, B1 // 128, 128), jnp.int32),
    mesh=_MESH,
    compiler_params=_SC_PARAMS,
    scratch_types=[
        pltpu.VMEM((CHUNK_ROWS, COLS), jnp.float32),
        pltpu.VMEM((CHUNK_ROWS, COLS), jnp.float32),
        pltpu.VMEM((B1 // 128, 128), jnp.int32),
        pltpu.SemaphoreType.DMA,
        pltpu.SemaphoreType.DMA,
    ],
)
def _hist1_call(loss_hbm, out_cnt, buf0, buf1, hist, sem0, sem1):
    wid = _wid()
    row0 = wid * SHARD_ROWS
    _zero2d_i32(hist, B1 // 128)
    ones = jnp.ones((16,), jnp.int32)

    def process(buf, carry):
        def qbody(q):
            r = lax.shift_right_logical(q, 2)
            c0 = (q & 3) * 128
            for u in range(8):
                v = buf[r, pl.ds(c0 + u * 16, 16)]
                bits = plsc.bitcast(v, jnp.int32)
                b = lax.shift_right_logical(bits, 16)
                plsc.addupdate_scatter(
                    hist, [lax.shift_right_logical(b, 7), b & 127], ones)
        plsc.parallel_loop(0, CHUNK_ROWS * 4, 1, unroll=4)(qbody)
        return carry

    _stream_chunks(loss_hbm, (buf0, buf1), (sem0, sem1), row0, process, 0)
    pltpu.sync_copy(hist, out_cnt.at[wid])


@functools.partial(
    pl.kernel,
    out_type=(jax.ShapeDtypeStruct((NW, B2 // 128, 128), jnp.int32),
              jax.ShapeDtypeStruct((NW, 16), jnp.float32)),
    mesh=_MESH,
    compiler_params=_SC_PARAMS,
    scratch_types=[
        pltpu.VMEM((CHUNK_ROWS, COLS), jnp.float32),
        pltpu.VMEM((CHUNK_ROWS, COLS), jnp.float32),
        pltpu.VMEM((B2 // 128, 128), jnp.int32),
        pltpu.VMEM((16,), jnp.int32),
        pltpu.VMEM((16,), jnp.float32),
        pltpu.SemaphoreType.DMA,
        pltpu.SemaphoreType.DMA,
    ],
)
def _hist2_call(loss_hbm, d1_hbm, out_cnt, out_sgt,
                buf0, buf1, hist, dvec, acc, sem0, sem1):
    wid = _wid()
    row0 = wid * SHARD_ROWS
    pltpu.sync_copy(d1_hbm, dvec)
    dv = dvec[...]
    b1 = dv[0]
    _zero2d_i32(hist, B2 // 128)
    ones = jnp.ones((16,), jnp.int32)
    zeros_f = jnp.zeros((16,), jnp.float32)

    def process(buf, carry):
        def qbody(q, cs):
            cs = list(cs)
            r = lax.shift_right_logical(q, 2)
            c0 = (q & 3) * 128
            for u in range(8):
                v = buf[r, pl.ds(c0 + u * 16, 16)]
                bits = plsc.bitcast(v, jnp.int32)
                hi = lax.shift_right_logical(bits, 16)
                low = bits & (B2 - 1)
                plsc.addupdate_scatter(
                    hist, [lax.shift_right_logical(low, 7), low & 127],
                    ones, mask=hi == b1)
                cs[u % 4] = cs[u % 4] + jnp.where(hi > b1, v, zeros_f)
            return tuple(cs)
        return plsc.parallel_loop(0, CHUNK_ROWS * 4, 1, unroll=3,
                                  carry=carry)(qbody)

    z4 = (jnp.zeros((16,), jnp.float32),) * 4
    accs = _stream_chunks(loss_hbm, (buf0, buf1), (sem0, sem1), row0, process,
                          z4)
    acc[...] = accs[0] + accs[1] + accs[2] + accs[3]
    pltpu.sync_copy(hist, out_cnt.at[wid])
    pltpu.sync_copy(acc, out_sgt.at[wid])


def _suffix_incl(g, rows_n):
    c1 = lax.broadcasted_iota(jnp.int32, (128, 128), 0)
    c2 = lax.broadcasted_iota(jnp.int32, (128, 128), 1)
    m = (c1 >= c2).astype(jnp.float32)
    sw = jax.lax.dot(g, m)
    rows = jnp.sum(g, axis=1)
    i1 = lax.broadcasted_iota(jnp.int32, (rows_n, rows_n), 0)
    i2 = lax.broadcasted_iota(jnp.int32, (rows_n, rows_n), 1)
    r2 = jnp.broadcast_to(rows[None, :], (rows_n, rows_n))
    sr = jnp.sum(jnp.where(i2 > i1, r2, 0.0), axis=1)
    return sw + sr[:, None]


def _flat_iota(rows_n):
    return (lax.broadcasted_iota(jnp.int32, (rows_n, 128), 0) * 128
            + lax.broadcasted_iota(jnp.int32, (rows_n, 128), 1))


def _decide1_body(cnt_ref, o_ref):
    g = jnp.sum(cnt_ref[...], axis=0)
    s = _suffix_incl(g.astype(jnp.float32), B1 // 128)
    b1 = jnp.sum((s >= jnp.float32(K_KEPT)).astype(jnp.int32)) - 1
    fi = _flat_iota(B1 // 128)
    n_above = jnp.sum(jnp.where(fi > b1, g, 0))
    r1 = K_KEPT - n_above
    idx = lax.broadcasted_iota(jnp.int32, (16,), 0)
    o_ref[...] = jnp.where(idx == 0, b1, jnp.where(idx == 1, r1, 0))


_decide1_call = pl.pallas_call(
    _decide1_body,
    out_shape=jax.ShapeDtypeStruct((16,), jnp.int32),
)


def _final_body(cnt_ref, sgt_ref, d1_ref, o_ref):
    b1 = d1_ref[0]
    r1 = d1_ref[1]
    g = jnp.sum(cnt_ref[...], axis=0)
    gf = g.astype(jnp.float32)
    s = _suffix_incl(gf, B2 // 128)
    b2 = jnp.sum((s >= r1.astype(jnp.float32)).astype(jnp.int32)) - 1
    fi = _flat_iota(B2 // 128)
    n_above = jnp.sum(jnp.where(fi > b2, g, 0))
    m = r1 - n_above
    vj = lax.bitcast_convert_type((b1 << 16) | fi, jnp.float32)
    sum_above = jnp.sum(jnp.where(fi > b2, gf, 0.0) * vj)
    tau = jnp.sum(jnp.where(fi == b2, vj, 0.0))
    total = jnp.sum(sgt_ref[...]) + sum_above + m.astype(jnp.float32) * tau
    o_ref[...] = (total * jnp.float32(1.0 / K_KEPT))[None]


_final_call = pl.pallas_call(
    _final_body,
    out_shape=jax.ShapeDtypeStruct((1,), jnp.float32),
)


def kernel(pred, target):
    x = pred.reshape(ROWS, COLS)
    t = target.reshape(ROWS, COLS)
    loss = _loss_call(x, t)
    cnt1 = _hist1_call(loss)
    d1 = _decide1_call(cnt1)
    cnt2, sgt = _hist2_call(loss, d1)
    out = _final_call(cnt2, sgt, d1)
    return out.reshape(())


# speedup vs baseline: 53.1937x; 1.0142x over previous
import functools

import jax
import jax.numpy as jnp
from jax import lax
from jax.experimental import pallas as pl
from jax.experimental.pallas import tpu as pltpu
from jax.experimental.pallas import tpu_sc as plsc

N = 16 * 1 * 512 * 512
K_KEPT = max(int(N * (1.0 - 0.7)), max(1, 10000))

NC, NS = 2, 16
NW = NC * NS
ROWS, COLS = 8192, 512
SHARD_ROWS = ROWS // NW
CHUNK_ROWS = 32
NCHUNK = SHARD_ROWS // CHUNK_ROWS
CHUNK_ROWS1 = 64
NCHUNK1 = SHARD_ROWS // CHUNK_ROWS1
U = 8

B1 = 32768
B2 = 65536

_MESH = plsc.VectorSubcoreMesh(core_axis_name="c", subcore_axis_name="s")
_SC_PARAMS = pltpu.CompilerParams(needs_layout_passes=False)


def _wid():
    return lax.axis_index("s") * NC + lax.axis_index("c")


def _zero2d_i32(ref, nrows):
    def body(r, _):
        for u in range(8):
            ref[r, pl.ds(u * 16, 16)] = jnp.zeros((16,), jnp.int32)
        return 0
    lax.fori_loop(0, nrows, body, 0)


def _stream_chunks(loss_hbm, bufs, sems, row0, process, carry,
                   chunk_rows=CHUNK_ROWS, nchunk=NCHUNK):
    cp = pltpu.async_copy(loss_hbm.at[pl.ds(row0, chunk_rows)], bufs[0], sems[0])
    for i in range(nchunk):
        if i + 1 < nchunk:
            nxt = pltpu.async_copy(
                loss_hbm.at[pl.ds(row0 + (i + 1) * chunk_rows, chunk_rows)],
                bufs[(i + 1) % 2], sems[(i + 1) % 2])
        cp.wait()
        carry = process(bufs[i % 2], carry)
        if i + 1 < nchunk:
            cp = nxt
    return carry


def _loss_body(x_ref, t_ref, o_ref):
    x = x_ref[...]
    t = t_ref[...]
    o_ref[...] = (jnp.maximum(x, 0.0) - x * t
                  + jnp.log1p(jnp.exp(-jnp.abs(x))))


_loss_call = pl.pallas_call(
    _loss_body,
    out_shape=jax.ShapeDtypeStruct((ROWS, COLS), jnp.float32),
    grid=(16,),
    in_specs=[pl.BlockSpec((ROWS // 16, COLS), lambda i: (i, 0)),
              pl.BlockSpec((ROWS // 16, COLS), lambda i: (i, 0))],
    out_specs=pl.BlockSpec((ROWS // 16, COLS), lambda i: (i, 0)),
)


@functools.partial(
    pl.kernel,
    out_type=jax.ShapeDtypeStruct((NW, B1 // 128, 128), jnp.int32),
    mesh=_MESH,
    compiler_params=_SC_PARAMS,
    scratch_types=[
        pltpu.VMEM((CHUNK_ROWS1, COLS), jnp.float32),
        pltpu.VMEM((CHUNK_ROWS1, COLS), jnp.float32),
        pltpu.VMEM((B1 // 128, 128), jnp.int32),
        pltpu.SemaphoreType.DMA,
        pltpu.SemaphoreType.DMA,
    ],
)
def _hist1_call(loss_hbm, out_cnt, buf0, buf1, hist, sem0, sem1):
    wid = _wid()
    row0 = wid * SHARD_ROWS
    _zero2d_i32(hist, B1 // 128)
    ones = jnp.ones((16,), jnp.int32)

    def process(buf, carry):
        def qbody(q):
            r = lax.shift_right_logical(q, 2)
            c0 = (q & 3) * 128
            for u in range(8):
                v = buf[r, pl.ds(c0 + u * 16, 16)]
                bits = plsc.bitcast(v, jnp.int32)
                b = lax.shift_right_logical(bits, 16)
                plsc.addupdate_scatter(
                    hist, [lax.shift_right_logical(b, 7), b & 127], ones)
        plsc.parallel_loop(0, CHUNK_ROWS1 * 4, 1, unroll=4)(qbody)
        return carry

    _stream_chunks(loss_hbm, (buf0, buf1), (sem0, sem1), row0, process, 0,
                   chunk_rows=CHUNK_ROWS1, nchunk=NCHUNK1)
    pltpu.sync_copy(hist, out_cnt.at[wid])


@functools.partial(
    pl.kernel,
    out_type=(jax.ShapeDtypeStruct((NW, B2 // 128, 128), jnp.int32),
              jax.ShapeDtypeStruct((NW, 16), jnp.float32)),
    mesh=_MESH,
    compiler_params=_SC_PARAMS,
    scratch_types=[
        pltpu.VMEM((CHUNK_ROWS, COLS), jnp.float32),
        pltpu.VMEM((CHUNK_ROWS, COLS), jnp.float32),
        pltpu.VMEM((B2 // 128, 128), jnp.int32),
        pltpu.VMEM((16,), jnp.int32),
        pltpu.VMEM((16,), jnp.float32),
        pltpu.SemaphoreType.DMA,
        pltpu.SemaphoreType.DMA,
    ],
)
def _hist2_call(loss_hbm, d1_hbm, out_cnt, out_sgt,
                buf0, buf1, hist, dvec, acc, sem0, sem1):
    wid = _wid()
    row0 = wid * SHARD_ROWS
    pltpu.sync_copy(d1_hbm, dvec)
    dv = dvec[...]
    b1 = dv[0]
    _zero2d_i32(hist, B2 // 128)
    ones = jnp.ones((16,), jnp.int32)
    zeros_f = jnp.zeros((16,), jnp.float32)

    def process(buf, carry):
        def qbody(q, cs):
            cs = list(cs)
            r = lax.shift_right_logical(q, 2)
            c0 = (q & 3) * 128
            for u in range(8):
                v = buf[r, pl.ds(c0 + u * 16, 16)]
                bits = plsc.bitcast(v, jnp.int32)
                hi = lax.shift_right_logical(bits, 16)
                low = bits & (B2 - 1)
                plsc.addupdate_scatter(
                    hist, [lax.shift_right_logical(low, 7), low & 127],
                    ones, mask=hi == b1)
                cs[u % 4] = cs[u % 4] + jnp.where(hi > b1, v, zeros_f)
            return tuple(cs)
        return plsc.parallel_loop(0, CHUNK_ROWS * 4, 1, unroll=3,
                                  carry=carry)(qbody)

    z4 = (jnp.zeros((16,), jnp.float32),) * 4
    accs = _stream_chunks(loss_hbm, (buf0, buf1), (sem0, sem1), row0, process,
                          z4)
    acc[...] = accs[0] + accs[1] + accs[2] + accs[3]
    pltpu.sync_copy(hist, out_cnt.at[wid])
    pltpu.sync_copy(acc, out_sgt.at[wid])


def _suffix_incl(g, rows_n):
    c1 = lax.broadcasted_iota(jnp.int32, (128, 128), 0)
    c2 = lax.broadcasted_iota(jnp.int32, (128, 128), 1)
    m = (c1 >= c2).astype(jnp.float32)
    sw = jax.lax.dot(g, m)
    rows = jnp.sum(g, axis=1)
    i1 = lax.broadcasted_iota(jnp.int32, (rows_n, rows_n), 0)
    i2 = lax.broadcasted_iota(jnp.int32, (rows_n, rows_n), 1)
    r2 = jnp.broadcast_to(rows[None, :], (rows_n, rows_n))
    sr = jnp.sum(jnp.where(i2 > i1, r2, 0.0), axis=1)
    return sw + sr[:, None]


def _flat_iota(rows_n):
    return (lax.broadcasted_iota(jnp.int32, (rows_n, 128), 0) * 128
            + lax.broadcasted_iota(jnp.int32, (rows_n, 128), 1))


def _decide1_body(cnt_ref, o_ref):
    g = jnp.sum(cnt_ref[...], axis=0)
    s = _suffix_incl(g.astype(jnp.float32), B1 // 128)
    b1 = jnp.sum((s >= jnp.float32(K_KEPT)).astype(jnp.int32)) - 1
    fi = _flat_iota(B1 // 128)
    n_above = jnp.sum(jnp.where(fi > b1, g, 0))
    r1 = K_KEPT - n_above
    idx = lax.broadcasted_iota(jnp.int32, (16,), 0)
    o_ref[...] = jnp.where(idx == 0, b1, jnp.where(idx == 1, r1, 0))


_decide1_call = pl.pallas_call(
    _decide1_body,
    out_shape=jax.ShapeDtypeStruct((16,), jnp.int32),
)


def _final_body(cnt_ref, sgt_ref, d1_ref, o_ref):
    b1 = d1_ref[0]
    r1 = d1_ref[1]
    g = jnp.sum(cnt_ref[...], axis=0)
    gf = g.astype(jnp.float32)
    s = _suffix_incl(gf, B2 // 128)
    b2 = jnp.sum((s >= r1.astype(jnp.float32)).astype(jnp.int32)) - 1
    fi = _flat_iota(B2 // 128)
    n_above = jnp.sum(jnp.where(fi > b2, g, 0))
    m = r1 - n_above
    vj = lax.bitcast_convert_type((b1 << 16) | fi, jnp.float32)
    sum_above = jnp.sum(jnp.where(fi > b2, gf, 0.0) * vj)
    tau = jnp.sum(jnp.where(fi == b2, vj, 0.0))
    total = jnp.sum(sgt_ref[...]) + sum_above + m.astype(jnp.float32) * tau
    o_ref[...] = (total * jnp.float32(1.0 / K_KEPT))[None]


_final_call = pl.pallas_call(
    _final_body,
    out_shape=jax.ShapeDtypeStruct((1,), jnp.float32),
)


def kernel(pred, target):
    x = pred.reshape(ROWS, COLS)
    t = target.reshape(ROWS, COLS)
    loss = _loss_call(x, t)
    cnt1 = _hist1_call(loss)
    d1 = _decide1_call(cnt1)
    cnt2, sgt = _hist2_call(loss, d1)
    out = _final_call(cnt2, sgt, d1)
    return out.reshape(())


# speedup vs baseline: 54.6780x; 1.0279x over previous
import functools

import jax
import jax.numpy as jnp
from jax import lax
from jax.experimental import pallas as pl
from jax.experimental.pallas import tpu as pltpu
from jax.experimental.pallas import tpu_sc as plsc

N = 16 * 1 * 512 * 512
K_KEPT = max(int(N * (1.0 - 0.7)), max(1, 10000))

NC, NS = 2, 16
NW = NC * NS
ROWS, COLS = 8192, 512
SHARD_ROWS = ROWS // NW
CHUNK_ROWS = 32
NCHUNK = SHARD_ROWS // CHUNK_ROWS
CHUNK_ROWS1 = 64
NCHUNK1 = SHARD_ROWS // CHUNK_ROWS1
U = 8

B1 = 32768
B2 = 65536

_MESH = plsc.VectorSubcoreMesh(core_axis_name="c", subcore_axis_name="s")
_SC_PARAMS = pltpu.CompilerParams(needs_layout_passes=False)


def _wid():
    return lax.axis_index("s") * NC + lax.axis_index("c")


def _zero2d_i32(ref, nrows):
    def body(r, _):
        for u in range(8):
            ref[r, pl.ds(u * 16, 16)] = jnp.zeros((16,), jnp.int32)
        return 0
    lax.fori_loop(0, nrows, body, 0)


def _stream_chunks(loss_hbm, bufs, sems, row0, process, carry,
                   chunk_rows=CHUNK_ROWS, nchunk=NCHUNK, prologue=None):
    cp = pltpu.async_copy(loss_hbm.at[pl.ds(row0, chunk_rows)], bufs[0], sems[0])
    if prologue is not None:
        prologue()
    for i in range(nchunk):
        if i + 1 < nchunk:
            nxt = pltpu.async_copy(
                loss_hbm.at[pl.ds(row0 + (i + 1) * chunk_rows, chunk_rows)],
                bufs[(i + 1) % 2], sems[(i + 1) % 2])
        cp.wait()
        carry = process(bufs[i % 2], carry)
        if i + 1 < nchunk:
            cp = nxt
    return carry


def _loss_body(x_ref, t_ref, o_ref):
    x = x_ref[...]
    t = t_ref[...]
    o_ref[...] = (jnp.maximum(x, 0.0) - x * t
                  + jnp.log1p(jnp.exp(-jnp.abs(x))))


_loss_call = pl.pallas_call(
    _loss_body,
    out_shape=jax.ShapeDtypeStruct((ROWS, COLS), jnp.float32),
    grid=(16,),
    in_specs=[pl.BlockSpec((ROWS // 16, COLS), lambda i: (i, 0)),
              pl.BlockSpec((ROWS // 16, COLS), lambda i: (i, 0))],
    out_specs=pl.BlockSpec((ROWS // 16, COLS), lambda i: (i, 0)),
)


@functools.partial(
    pl.kernel,
    out_type=jax.ShapeDtypeStruct((NW, B1 // 128, 128), jnp.int32),
    mesh=_MESH,
    compiler_params=_SC_PARAMS,
    scratch_types=[
        pltpu.VMEM((CHUNK_ROWS1, COLS), jnp.float32),
        pltpu.VMEM((CHUNK_ROWS1, COLS), jnp.float32),
        pltpu.VMEM((B1 // 128, 128), jnp.int32),
        pltpu.SemaphoreType.DMA,
        pltpu.SemaphoreType.DMA,
    ],
)
def _hist1_call(loss_hbm, out_cnt, buf0, buf1, hist, sem0, sem1):
    wid = _wid()
    row0 = wid * SHARD_ROWS
    ones = jnp.ones((16,), jnp.int32)

    def process(buf, carry):
        def qbody(q):
            r = lax.shift_right_logical(q, 2)
            c0 = (q & 3) * 128
            for u in range(8):
                v = buf[r, pl.ds(c0 + u * 16, 16)]
                bits = plsc.bitcast(v, jnp.int32)
                b = lax.shift_right_logical(bits, 16)
                plsc.addupdate_scatter(
                    hist, [lax.shift_right_logical(b, 7), b & 127], ones)
        plsc.parallel_loop(0, CHUNK_ROWS1 * 4, 1, unroll=4)(qbody)
        return carry

    _stream_chunks(loss_hbm, (buf0, buf1), (sem0, sem1), row0, process, 0,
                   chunk_rows=CHUNK_ROWS1, nchunk=NCHUNK1,
                   prologue=lambda: _zero2d_i32(hist, B1 // 128))
    pltpu.sync_copy(hist, out_cnt.at[wid])


@functools.partial(
    pl.kernel,
    out_type=(jax.ShapeDtypeStruct((NW, B2 // 128, 128), jnp.int32),
              jax.ShapeDtypeStruct((NW, 16), jnp.float32)),
    mesh=_MESH,
    compiler_params=_SC_PARAMS,
    scratch_types=[
        pltpu.VMEM((CHUNK_ROWS, COLS), jnp.float32),
        pltpu.VMEM((CHUNK_ROWS, COLS), jnp.float32),
        pltpu.VMEM((B2 // 128, 128), jnp.int32),
        pltpu.VMEM((16,), jnp.int32),
        pltpu.VMEM((16,), jnp.float32),
        pltpu.SemaphoreType.DMA,
        pltpu.SemaphoreType.DMA,
    ],
)
def _hist2_call(loss_hbm, d1_hbm, out_cnt, out_sgt,
                buf0, buf1, hist, dvec, acc, sem0, sem1):
    wid = _wid()
    row0 = wid * SHARD_ROWS
    pltpu.sync_copy(d1_hbm, dvec)
    dv = dvec[...]
    b1 = dv[0]
    ones = jnp.ones((16,), jnp.int32)
    zeros_f = jnp.zeros((16,), jnp.float32)

    def process(buf, carry):
        def qbody(q, cs):
            cs = list(cs)
            r = lax.shift_right_logical(q, 2)
            c0 = (q & 3) * 128
            for u in range(8):
                v = buf[r, pl.ds(c0 + u * 16, 16)]
                bits = plsc.bitcast(v, jnp.int32)
                hi = lax.shift_right_logical(bits, 16)
                low = bits & (B2 - 1)
                plsc.addupdate_scatter(
                    hist, [lax.shift_right_logical(low, 7), low & 127],
                    ones, mask=hi == b1)
                cs[u % 4] = cs[u % 4] + jnp.where(hi > b1, v, zeros_f)
            return tuple(cs)
        return plsc.parallel_loop(0, CHUNK_ROWS * 4, 1, unroll=3,
                                  carry=carry)(qbody)

    z4 = (jnp.zeros((16,), jnp.float32),) * 4
    accs = _stream_chunks(loss_hbm, (buf0, buf1), (sem0, sem1), row0, process,
                          z4, prologue=lambda: _zero2d_i32(hist, B2 // 128))
    acc[...] = accs[0] + accs[1] + accs[2] + accs[3]
    pltpu.sync_copy(hist, out_cnt.at[wid])
    pltpu.sync_copy(acc, out_sgt.at[wid])


def _suffix_incl(g, rows_n):
    c1 = lax.broadcasted_iota(jnp.int32, (128, 128), 0)
    c2 = lax.broadcasted_iota(jnp.int32, (128, 128), 1)
    m = (c1 >= c2).astype(jnp.float32)
    sw = jax.lax.dot(g, m)
    rows = jnp.sum(g, axis=1)
    i1 = lax.broadcasted_iota(jnp.int32, (rows_n, rows_n), 0)
    i2 = lax.broadcasted_iota(jnp.int32, (rows_n, rows_n), 1)
    r2 = jnp.broadcast_to(rows[None, :], (rows_n, rows_n))
    sr = jnp.sum(jnp.where(i2 > i1, r2, 0.0), axis=1)
    return sw + sr[:, None]


def _flat_iota(rows_n):
    return (lax.broadcasted_iota(jnp.int32, (rows_n, 128), 0) * 128
            + lax.broadcasted_iota(jnp.int32, (rows_n, 128), 1))


def _decide1_body(cnt_ref, o_ref):
    g = jnp.sum(cnt_ref[...], axis=0)
    s = _suffix_incl(g.astype(jnp.float32), B1 // 128)
    b1 = jnp.sum((s >= jnp.float32(K_KEPT)).astype(jnp.int32)) - 1
    fi = _flat_iota(B1 // 128)
    n_above = jnp.sum(jnp.where(fi > b1, g, 0))
    r1 = K_KEPT - n_above
    idx = lax.broadcasted_iota(jnp.int32, (16,), 0)
    o_ref[...] = jnp.where(idx == 0, b1, jnp.where(idx == 1, r1, 0))


_decide1_call = pl.pallas_call(
    _decide1_body,
    out_shape=jax.ShapeDtypeStruct((16,), jnp.int32),
)


def _final_body(cnt_ref, sgt_ref, d1_ref, o_ref):
    b1 = d1_ref[0]
    r1 = d1_ref[1]
    g = jnp.sum(cnt_ref[...], axis=0)
    gf = g.astype(jnp.float32)
    s = _suffix_incl(gf, B2 // 128)
    b2 = jnp.sum((s >= r1.astype(jnp.float32)).astype(jnp.int32)) - 1
    fi = _flat_iota(B2 // 128)
    n_above = jnp.sum(jnp.where(fi > b2, g, 0))
    m = r1 - n_above
    vj = lax.bitcast_convert_type((b1 << 16) | fi, jnp.float32)
    sum_above = jnp.sum(jnp.where(fi > b2, gf, 0.0) * vj)
    tau = jnp.sum(jnp.where(fi == b2, vj, 0.0))
    total = jnp.sum(sgt_ref[...]) + sum_above + m.astype(jnp.float32) * tau
    o_ref[...] = (total * jnp.float32(1.0 / K_KEPT))[None]


_final_call = pl.pallas_call(
    _final_body,
    out_shape=jax.ShapeDtypeStruct((1,), jnp.float32),
)


def kernel(pred, target):
    x = pred.reshape(ROWS, COLS)
    t = target.reshape(ROWS, COLS)
    loss = _loss_call(x, t)
    cnt1 = _hist1_call(loss)
    d1 = _decide1_call(cnt1)
    cnt2, sgt = _hist2_call(loss, d1)
    out = _final_call(cnt2, sgt, d1)
    return out.reshape(())


# speedup vs baseline: 55.1748x; 1.0091x over previous
import functools

import jax
import jax.numpy as jnp
from jax import lax
from jax.experimental import pallas as pl
from jax.experimental.pallas import tpu as pltpu
from jax.experimental.pallas import tpu_sc as plsc

N = 16 * 1 * 512 * 512
K_KEPT = max(int(N * (1.0 - 0.7)), max(1, 10000))

NC, NS = 2, 16
NW = NC * NS
ROWS, COLS = 8192, 512
SHARD_ROWS = ROWS // NW
CHUNK_ROWS = 32
NCHUNK = SHARD_ROWS // CHUNK_ROWS
CHUNK_ROWS1 = 64
NCHUNK1 = SHARD_ROWS // CHUNK_ROWS1
U = 8

B1 = 32768
B2 = 65536

_MESH = plsc.VectorSubcoreMesh(core_axis_name="c", subcore_axis_name="s")
_SC_PARAMS = pltpu.CompilerParams(needs_layout_passes=False)


def _wid():
    return lax.axis_index("s") * NC + lax.axis_index("c")


def _zero2d_i32(ref, nrows):
    def body(r, _):
        for u in range(8):
            ref[r, pl.ds(u * 16, 16)] = jnp.zeros((16,), jnp.int32)
        return 0
    lax.fori_loop(0, nrows, body, 0)


def _stream_chunks(loss_hbm, bufs, sems, row0, process, carry,
                   chunk_rows=CHUNK_ROWS, nchunk=NCHUNK, prologue=None):
    cp = pltpu.async_copy(loss_hbm.at[pl.ds(row0, chunk_rows)], bufs[0], sems[0])
    if prologue is not None:
        prologue()
    for i in range(nchunk):
        if i + 1 < nchunk:
            nxt = pltpu.async_copy(
                loss_hbm.at[pl.ds(row0 + (i + 1) * chunk_rows, chunk_rows)],
                bufs[(i + 1) % 2], sems[(i + 1) % 2])
        cp.wait()
        carry = process(bufs[i % 2], carry)
        if i + 1 < nchunk:
            cp = nxt
    return carry


def _loss_body(x_ref, t_ref, o_ref):
    x = x_ref[...]
    t = t_ref[...]
    o_ref[...] = (jnp.maximum(x, 0.0) - x * t
                  + jnp.log1p(jnp.exp(-jnp.abs(x))))


_loss_call = pl.pallas_call(
    _loss_body,
    out_shape=jax.ShapeDtypeStruct((ROWS, COLS), jnp.float32),
    grid=(16,),
    in_specs=[pl.BlockSpec((ROWS // 16, COLS), lambda i: (i, 0)),
              pl.BlockSpec((ROWS // 16, COLS), lambda i: (i, 0))],
    out_specs=pl.BlockSpec((ROWS // 16, COLS), lambda i: (i, 0)),
)


@functools.partial(
    pl.kernel,
    out_type=jax.ShapeDtypeStruct((NW, B1 // 128, 128), jnp.int32),
    mesh=_MESH,
    compiler_params=_SC_PARAMS,
    scratch_types=[
        pltpu.VMEM((CHUNK_ROWS1, COLS), jnp.float32),
        pltpu.VMEM((CHUNK_ROWS1, COLS), jnp.float32),
        pltpu.VMEM((B1 // 128, 128), jnp.int32),
        pltpu.SemaphoreType.DMA,
        pltpu.SemaphoreType.DMA,
    ],
)
def _hist1_call(loss_hbm, out_cnt, buf0, buf1, hist, sem0, sem1):
    wid = _wid()
    row0 = wid * SHARD_ROWS
    ones = jnp.ones((16,), jnp.int32)

    def process(buf, carry):
        def qbody(q):
            r = lax.shift_right_logical(q, 2)
            c0 = (q & 3) * 128
            for u in range(8):
                v = buf[r, pl.ds(c0 + u * 16, 16)]
                bits = plsc.bitcast(v, jnp.int32)
                b = lax.shift_right_logical(bits, 16)
                plsc.addupdate_scatter(
                    hist, [lax.shift_right_logical(b, 7), b & 127], ones)
        plsc.parallel_loop(0, CHUNK_ROWS1 * 4, 1, unroll=4)(qbody)
        return carry

    _stream_chunks(loss_hbm, (buf0, buf1), (sem0, sem1), row0, process, 0,
                   chunk_rows=CHUNK_ROWS1, nchunk=NCHUNK1,
                   prologue=lambda: _zero2d_i32(hist, B1 // 128))
    pltpu.sync_copy(hist, out_cnt.at[wid])


@functools.partial(
    pl.kernel,
    out_type=(jax.ShapeDtypeStruct((NW, B2 // 128, 128), jnp.int32),
              jax.ShapeDtypeStruct((NW, 16), jnp.float32)),
    mesh=_MESH,
    compiler_params=_SC_PARAMS,
    scratch_types=[
        pltpu.VMEM((CHUNK_ROWS, COLS), jnp.float32),
        pltpu.VMEM((CHUNK_ROWS, COLS), jnp.float32),
        pltpu.VMEM((B2 // 128, 128), jnp.int32),
        pltpu.VMEM((16,), jnp.int32),
        pltpu.VMEM((16,), jnp.float32),
        pltpu.SemaphoreType.DMA,
        pltpu.SemaphoreType.DMA,
    ],
)
def _hist2_call(loss_hbm, d1_hbm, out_cnt, out_sgt,
                buf0, buf1, hist, dvec, acc, sem0, sem1):
    wid = _wid()
    row0 = wid * SHARD_ROWS
    ones = jnp.ones((16,), jnp.int32)
    cell = {}

    def prologue():
        pltpu.sync_copy(d1_hbm, dvec)
        cell["b1"] = dvec[...][0]
        _zero2d_i32(hist, B2 // 128)
    zeros_f = jnp.zeros((16,), jnp.float32)

    def process(buf, carry):
        b1 = cell["b1"]

        def qbody(q, cs):
            cs = list(cs)
            r = lax.shift_right_logical(q, 2)
            c0 = (q & 3) * 128
            for u in range(8):
                v = buf[r, pl.ds(c0 + u * 16, 16)]
                bits = plsc.bitcast(v, jnp.int32)
                hi = lax.shift_right_logical(bits, 16)
                low = bits & (B2 - 1)
                plsc.addupdate_scatter(
                    hist, [lax.shift_right_logical(low, 7), low & 127],
                    ones, mask=hi == b1)
                cs[u % 4] = cs[u % 4] + jnp.where(hi > b1, v, zeros_f)
            return tuple(cs)
        return plsc.parallel_loop(0, CHUNK_ROWS * 4, 1, unroll=4,
                                  carry=carry)(qbody)

    z4 = (jnp.zeros((16,), jnp.float32),) * 4
    accs = _stream_chunks(loss_hbm, (buf0, buf1), (sem0, sem1), row0, process,
                          z4, prologue=prologue)
    acc[...] = accs[0] + accs[1] + accs[2] + accs[3]
    pltpu.sync_copy(hist, out_cnt.at[wid])
    pltpu.sync_copy(acc, out_sgt.at[wid])


def _suffix_incl(g, rows_n):
    c1 = lax.broadcasted_iota(jnp.int32, (128, 128), 0)
    c2 = lax.broadcasted_iota(jnp.int32, (128, 128), 1)
    m = (c1 >= c2).astype(jnp.float32)
    sw = jax.lax.dot(g, m)
    rows = jnp.sum(g, axis=1)
    i1 = lax.broadcasted_iota(jnp.int32, (rows_n, rows_n), 0)
    i2 = lax.broadcasted_iota(jnp.int32, (rows_n, rows_n), 1)
    r2 = jnp.broadcast_to(rows[None, :], (rows_n, rows_n))
    sr = jnp.sum(jnp.where(i2 > i1, r2, 0.0), axis=1)
    return sw + sr[:, None]


def _flat_iota(rows_n):
    return (lax.broadcasted_iota(jnp.int32, (rows_n, 128), 0) * 128
            + lax.broadcasted_iota(jnp.int32, (rows_n, 128), 1))


def _decide1_body(cnt_ref, o_ref):
    g = jnp.sum(cnt_ref[...], axis=0)
    s = _suffix_incl(g.astype(jnp.float32), B1 // 128)
    b1 = jnp.sum((s >= jnp.float32(K_KEPT)).astype(jnp.int32)) - 1
    fi = _flat_iota(B1 // 128)
    n_above = jnp.sum(jnp.where(fi > b1, g, 0))
    r1 = K_KEPT - n_above
    idx = lax.broadcasted_iota(jnp.int32, (16,), 0)
    o_ref[...] = jnp.where(idx == 0, b1, jnp.where(idx == 1, r1, 0))


_decide1_call = pl.pallas_call(
    _decide1_body,
    out_shape=jax.ShapeDtypeStruct((16,), jnp.int32),
)


def _final_body(cnt_ref, sgt_ref, d1_ref, o_ref):
    b1 = d1_ref[0]
    r1 = d1_ref[1]
    g = jnp.sum(cnt_ref[...], axis=0)
    gf = g.astype(jnp.float32)
    s = _suffix_incl(gf, B2 // 128)
    b2 = jnp.sum((s >= r1.astype(jnp.float32)).astype(jnp.int32)) - 1
    fi = _flat_iota(B2 // 128)
    n_above = jnp.sum(jnp.where(fi > b2, g, 0))
    m = r1 - n_above
    vj = lax.bitcast_convert_type((b1 << 16) | fi, jnp.float32)
    sum_above = jnp.sum(jnp.where(fi > b2, gf, 0.0) * vj)
    tau = jnp.sum(jnp.where(fi == b2, vj, 0.0))
    total = jnp.sum(sgt_ref[...]) + sum_above + m.astype(jnp.float32) * tau
    o_ref[...] = (total * jnp.float32(1.0 / K_KEPT))[None]


_final_call = pl.pallas_call(
    _final_body,
    out_shape=jax.ShapeDtypeStruct((1,), jnp.float32),
)


def kernel(pred, target):
    x = pred.reshape(ROWS, COLS)
    t = target.reshape(ROWS, COLS)
    loss = _loss_call(x, t)
    cnt1 = _hist1_call(loss)
    d1 = _decide1_call(cnt1)
    cnt2, sgt = _hist2_call(loss, d1)
    out = _final_call(cnt2, sgt, d1)
    return out.reshape(())
